# baseline (speedup 1.0000x reference)
import functools

import jax
import jax.numpy as jnp
from jax import lax
from jax.experimental import pallas as pl
from jax.experimental.pallas import tpu as pltpu
from jax.experimental.pallas import tpu_sc as plsc

_NC = 2
_NS = 16
_NW = _NC * _NS
_NB = 8
_A = 6
_K = 8
_PAD = 56


def _gather_body(ch, seq, embed, idx_hbm, table_hbm, out_hbm, idx_v, rows_v,
                 gsem, wsem):
    wid = lax.axis_index("s") * _NC + lax.axis_index("c")
    pltpu.sync_copy(idx_hbm.at[wid], idx_v)
    base = wid * ch

    def gather(c, b):
        return pltpu.make_async_copy(
            table_hbm.at[idx_v.at[c]], rows_v.at[pl.ds(b * _PAD, seq)],
            gsem.at[b])

    def write(c, b):
        return pltpu.make_async_copy(
            rows_v.at[pl.ds(b * _PAD, _PAD)], out_hbm.at[base + c],
            wsem.at[b])

    def step(c, b, do_wait_w, do_gather):
        gather(c, b).wait()
        write(c, b).start()
        f = c + _A
        bf = (b + _A) % _NB
        if do_wait_w:
            write(f - _NB, bf).wait()
        if do_gather:
            gather(f, bf).start()

    for r in range(_A):
        gather(r, r % _NB).start()

    for r in range(_NB):
        step(r, r, do_wait_w=(r + _A >= _NB), do_gather=True)

    def outer(j, carry):
        c0 = j * _NB
        for r in range(_NB):
            step(c0 + r, r, do_wait_w=True, do_gather=True)
        return carry

    lax.fori_loop(1, ch // _NB - 1, outer, 0)

    for r in range(_NB):
        step(ch - _NB + r, r, do_wait_w=(r + _A < _NB),
             do_gather=(r + _A < _NB))

    for b in range(_NB):
        write(ch - _NB + b, b).wait()


def _sc_gather_slab(idx, table, ch, seq, embed):
    mesh = plsc.VectorSubcoreMesh(core_axis_name="c", subcore_axis_name="s")
    fn = pl.kernel(
        functools.partial(_gather_body, ch, seq, embed),
        mesh=mesh,
        out_type=jax.ShapeDtypeStruct((_NW * ch, _PAD, embed), jnp.float32),
        scratch_types=[
            pltpu.VMEM((ch, seq), jnp.int32),
            pltpu.VMEM((_NB * _PAD, embed), jnp.float32),
            pltpu.SemaphoreType.DMA((_NB,)),
            pltpu.SemaphoreType.DMA((_NB,)),
        ],
    )
    return fn(idx, table)


_ROWS = 32


def _relayout_body(flat_ref, out_ref):
    out_ref[...] = flat_ref[:, :out_ref.shape[1], :]


def _tc_scatter_slab(big, padded, k, b, bk, seq, embed):
    grid = (bk // _ROWS,)
    blk0 = k * bk // _ROWS
    in_slab_spec = pl.BlockSpec((_ROWS, _PAD, embed), lambda g: (g, 0, 0))
    out_spec = pl.BlockSpec(
        (_ROWS, seq, embed), lambda g, _b=blk0: (_b + g, 0, 0))
    out_shape = jax.ShapeDtypeStruct((b, seq, embed), jnp.float32)
    if big is None:
        return pl.pallas_call(
            _relayout_body,
            grid=grid,
            in_specs=[in_slab_spec],
            out_specs=out_spec,
            out_shape=out_shape,
        )(padded)

    def kbody(big_ref, flat_ref, out_ref):
        del big_ref
        _relayout_body(flat_ref, out_ref)

    return pl.pallas_call(
        kbody,
        grid=grid,
        in_specs=[pl.BlockSpec(memory_space=pl.ANY), in_slab_spec],
        out_specs=out_spec,
        out_shape=out_shape,
        input_output_aliases={0: 0},
    )(big, padded)


def kernel(X, embedding_weight):
    b, s = X.shape
    vocab, embed = embedding_weight.shape
    bk = b // _K
    ch = bk // _NW
    idx = X.reshape(_K, _NW, ch, s).astype(jnp.int32)
    out = None
    for k in range(_K):
        padded = _sc_gather_slab(idx[k], embedding_weight, ch, s, embed)
        out = _tc_scatter_slab(out, padded, k, b, bk, s, embed)
    return out
